# baseline (speedup 1.0000x reference)
import functools
import jax
import jax.numpy as jnp
from jax import lax
from jax.experimental import pallas as pl
from jax.experimental.pallas import tpu as pltpu
from jax.experimental.pallas import tpu_sc as plsc

H = 512
W = 512
C = 17
P = H * W
NW = 32
PPW = P // NW
CH = 256
ROUNDS = PPW // CH
L = 16


def _sc_body(hm_hbm, off_hbm, out_hbm, hm_v, off_v, out_v):
    wid = lax.axis_index("s") * 2 + lax.axis_index("c")
    iota = lax.iota(jnp.int32, L)

    for r in range(ROUNDS):
        base_p = wid * PPW + r * CH
        pltpu.sync_copy(hm_hbm.at[pl.ds(base_p, CH)], hm_v)
        pltpu.sync_copy(off_hbm.at[pl.ds(base_p, CH)], off_v)

        def body(j, _):
            p_loc = j * L + iota
            c0 = jnp.zeros((L,), jnp.int32)
            best = plsc.load_gather(hm_v, [p_loc, c0])
            bestc = c0
            for c in range(1, C):
                v = plsc.load_gather(hm_v, [p_loc, c0 + c])
                gt = v > best
                best = jnp.where(gt, v, best)
                bestc = jnp.where(gt, jnp.full((L,), c, jnp.int32), bestc)
            score = 1.0 / (1.0 + jnp.exp(-best))

            y_off = plsc.load_gather(off_v, [p_loc, bestc])
            x_off = plsc.load_gather(off_v, [p_loc, bestc + C])

            p_glob = base_p + p_loc
            py = (p_glob >> 9).astype(jnp.float32)
            px = (p_glob & (W - 1)).astype(jnp.float32)
            xv = (px * 4.0 + x_off).astype(jnp.int32).astype(jnp.float32)
            yv = (py * 4.0 + y_off).astype(jnp.int32).astype(jnp.float32)

            plsc.store_scatter(out_v, [p_loc, c0], bestc.astype(jnp.float32))
            plsc.store_scatter(out_v, [p_loc, c0 + 1], score)
            plsc.store_scatter(out_v, [p_loc, c0 + 2], xv)
            plsc.store_scatter(out_v, [p_loc, c0 + 3], yv)
            return 0

        lax.fori_loop(0, CH // L, body, 0)
        pltpu.sync_copy(out_v, out_hbm.at[pl.ds(base_p, CH)])


def kernel(heatmaps_input, offsets_input):
    hm = heatmaps_input.reshape(P, C)
    off = offsets_input.reshape(P, 2 * C)
    k = functools.partial(
        pl.kernel,
        out_type=jax.ShapeDtypeStruct((P, 4), jnp.float32),
        scratch_types=[
            pltpu.VMEM((CH, C), jnp.float32),
            pltpu.VMEM((CH, 2 * C), jnp.float32),
            pltpu.VMEM((CH, 4), jnp.float32),
        ],
        mesh=plsc.VectorSubcoreMesh(core_axis_name="c", subcore_axis_name="s"),
        compiler_params=pltpu.CompilerParams(
            use_tc_tiling_on_sc=False, needs_layout_passes=False
        ),
    )(_sc_body)
    out = k(hm, off)
    return out.reshape(1, P, 4)


# speedup vs baseline: 1.5417x; 1.5417x over previous
import functools
import jax
import jax.numpy as jnp
from jax import lax
from jax.experimental import pallas as pl
from jax.experimental.pallas import tpu as pltpu
from jax.experimental.pallas import tpu_sc as plsc

H = 512
W = 512
C = 17
P = H * W
NW = 32
PPW = P // NW
CH = 512
ROUNDS = PPW // CH
L = 16


def _sc_body(hm_hbm, off_hbm, out_hbm, hm_v, off_v, out_v):
    wid = lax.axis_index("s") * 2 + lax.axis_index("c")
    iota = lax.iota(jnp.int32, L)
    iota17 = iota * C
    iota34 = iota * (2 * C)

    for r in range(ROUNDS):
        base_p = wid * PPW + r * CH
        pltpu.sync_copy(hm_hbm.at[pl.ds(base_p * C // 128, CH * C // 128)], hm_v)
        pltpu.sync_copy(
            off_hbm.at[pl.ds(base_p * 2 * C // 128, CH * 2 * C // 128)], off_v
        )

        def body(j, _):
            p_loc = j * L + iota
            hbase = j * (L * C) + iota17
            best = plsc.load_gather(hm_v, [hbase >> 7, hbase & 127])
            bestc = jnp.zeros((L,), jnp.int32)
            for c in range(1, C):
                q = hbase + c
                v = plsc.load_gather(hm_v, [q >> 7, q & 127])
                gt = v > best
                best = jnp.where(gt, v, best)
                bestc = jnp.where(gt, jnp.full((L,), c, jnp.int32), bestc)
            score = 1.0 / (1.0 + jnp.exp(-best))

            obase = j * (L * 2 * C) + iota34 + bestc
            ox = obase + C
            y_off = plsc.load_gather(off_v, [obase >> 7, obase & 127])
            x_off = plsc.load_gather(off_v, [ox >> 7, ox & 127])

            p_glob = base_p + p_loc
            py = (p_glob >> 9).astype(jnp.float32)
            px = (p_glob & (W - 1)).astype(jnp.float32)
            xv = (px * 4.0 + x_off).astype(jnp.int32).astype(jnp.float32)
            yv = (py * 4.0 + y_off).astype(jnp.int32).astype(jnp.float32)

            c0 = jnp.zeros((L,), jnp.int32)
            plsc.store_scatter(out_v, [p_loc, c0], bestc.astype(jnp.float32))
            plsc.store_scatter(out_v, [p_loc, c0 + 1], score)
            plsc.store_scatter(out_v, [p_loc, c0 + 2], xv)
            plsc.store_scatter(out_v, [p_loc, c0 + 3], yv)
            return 0

        lax.fori_loop(0, CH // L, body, 0)
        pltpu.sync_copy(out_v, out_hbm.at[0, pl.ds(base_p, CH)])


def kernel(heatmaps_input, offsets_input):
    hm = heatmaps_input.reshape(P * C // 128, 128)
    off = offsets_input.reshape(P * 2 * C // 128, 128)
    k = functools.partial(
        pl.kernel,
        out_type=jax.ShapeDtypeStruct((1, P, 4), jnp.float32),
        scratch_types=[
            pltpu.VMEM((CH * C // 128, 128), jnp.float32),
            pltpu.VMEM((CH * 2 * C // 128, 128), jnp.float32),
            pltpu.VMEM((CH, 4), jnp.float32),
        ],
        mesh=plsc.VectorSubcoreMesh(core_axis_name="c", subcore_axis_name="s"),
        compiler_params=pltpu.CompilerParams(
            use_tc_tiling_on_sc=False, needs_layout_passes=False
        ),
    )(_sc_body)
    return k(hm, off)


# speedup vs baseline: 2.2372x; 1.4511x over previous
import jax
import jax.numpy as jnp
from jax import lax
from jax.experimental import pallas as pl
from jax.experimental.pallas import tpu as pltpu

H = 512
W = 512
C = 17
BH = 16


def _decode_kernel(hm_ref, off_ref, out_ref):
    i = pl.program_id(0)
    hmv = hm_ref[...]
    offv = off_ref[...]

    m = jnp.max(hmv, axis=-1, keepdims=True)
    iota_c = lax.broadcasted_iota(jnp.int32, (BH, W, C), 2)
    a = (C - 1) - jnp.max(
        jnp.where(hmv == m, (C - 1) - iota_c, 0), axis=-1, keepdims=True
    )
    score = jax.nn.sigmoid(m)

    iota34 = lax.broadcasted_iota(jnp.int32, (BH, W, 2 * C), 2)
    y_off = jnp.sum(jnp.where(iota34 == a, offv, 0.0), axis=-1, keepdims=True)
    x_off = jnp.sum(jnp.where(iota34 == a + C, offv, 0.0), axis=-1, keepdims=True)

    row = (i * BH + lax.broadcasted_iota(jnp.int32, (BH, W, 1), 0)).astype(jnp.float32)
    col = lax.broadcasted_iota(jnp.int32, (BH, W, 1), 1).astype(jnp.float32)
    xv = (col * 4.0 + x_off).astype(jnp.int32).astype(jnp.float32)
    yv = (row * 4.0 + y_off).astype(jnp.int32).astype(jnp.float32)

    out_ref[...] = jnp.concatenate([a.astype(jnp.float32), score, xv, yv], axis=-1)


def kernel(heatmaps_input, offsets_input):
    hm = heatmaps_input.reshape(H, W, C)
    off = offsets_input.reshape(H, W, 2 * C)
    out = pl.pallas_call(
        _decode_kernel,
        grid=(H // BH,),
        in_specs=[
            pl.BlockSpec((BH, W, C), lambda i: (i, 0, 0)),
            pl.BlockSpec((BH, W, 2 * C), lambda i: (i, 0, 0)),
        ],
        out_specs=pl.BlockSpec((BH, W, 4), lambda i: (i, 0, 0)),
        out_shape=jax.ShapeDtypeStruct((H, W, 4), jnp.float32),
        compiler_params=pltpu.CompilerParams(
            dimension_semantics=("arbitrary",),
            vmem_limit_bytes=100 * 1024 * 1024,
        ),
    )(hm, off)
    return out.reshape(1, H * W, 4)
